# baseline (speedup 1.0000x reference)
import functools

import jax
import jax.numpy as jnp
import numpy as np
from jax.experimental import pallas as pl
from jax.experimental.pallas import tpu as pltpu

B = 16
T = 128
V = 25
M = 2
NUM_CLASSES = 27

SK_TG = 4
SK_GROUPS = M * T // SK_TG
SK_COLS = SK_GROUPS * 128
SK_VALID = M * T * V

IMU_TG = 16
IMU_GROUPS = T // IMU_TG
IMU_COLS = IMU_GROUPS * 128
IMU_VALID = T * 6


def _sk_kernel(x_ref, *refs):
    a_refs = refs[:10]
    w_refs = refs[10:20]
    out_ref = refs[20]
    x = x_ref[0]
    for l in range(10):
        c = x.shape[0]
        xr = x.reshape(c * SK_GROUPS, 128)
        y = jax.lax.dot_general(xr, a_refs[l][...],
                                (((1,), (0,)), ((), ())),
                                preferred_element_type=jnp.float32)
        y = y.reshape(c, SK_COLS).astype(jnp.bfloat16)
        z = jax.lax.dot_general(w_refs[l][...], y,
                                (((1,), (0,)), ((), ())),
                                preferred_element_type=jnp.float32)
        x = jnp.maximum(z.astype(jnp.bfloat16), jnp.bfloat16(0.0))
    out_ref[0, 0, :] = jnp.sum(x.astype(jnp.float32), axis=1) * (1.0 / SK_VALID)


def _imu_kernel(y_ref, a_ref, *refs):
    w_refs = refs[:5]
    out_ref = refs[5]
    y = y_ref[0]
    a = a_ref[...]
    for l in range(5):
        c = y.shape[0]
        yr = y.reshape(c * IMU_GROUPS, 128)
        s = jax.lax.dot_general(yr, a, (((1,), (0,)), ((), ())),
                                preferred_element_type=jnp.float32)
        s = s.reshape(c, IMU_COLS).astype(jnp.bfloat16)
        z = jax.lax.dot_general(w_refs[l][...], s,
                                (((1,), (0,)), ((), ())),
                                preferred_element_type=jnp.float32)
        y = jnp.maximum(z.astype(jnp.bfloat16), jnp.bfloat16(0.0))
    out_ref[0, 0, :] = jnp.sum(y.astype(jnp.float32), axis=1) * (1.0 / IMU_VALID)


def _fc_kernel(sk_ref, imu_ref, wt_ref, wb_ref, b_ref, out_ref):
    top = jax.lax.dot_general(sk_ref[...], wt_ref[...],
                              (((1,), (0,)), ((), ())),
                              preferred_element_type=jnp.float32)
    bot = jax.lax.dot_general(imu_ref[...], wb_ref[...],
                              (((1,), (0,)), ((), ())),
                              preferred_element_type=jnp.float32)
    out_ref[...] = top + bot + b_ref[...]


def _full(shape):
    ndim = len(shape)
    return pl.BlockSpec(shape, lambda *_: (0,) * ndim)


def kernel(skeleton, inertial, A_sk, A_imu, Ws_sk, Bs_sk, Ws_imu, W_fc, b_fc):
    f32 = jnp.float32

    eye4 = jnp.eye(SK_TG, dtype=f32)
    a_sk_packed = []
    for Badp in Bs_sk:
        ahat = A_sk + Badp
        a4 = jnp.kron(eye4, ahat)
        a_sk_packed.append(jnp.pad(a4, ((0, 28), (0, 28))).astype(jnp.bfloat16))
    wt_sk = [w.T.astype(jnp.bfloat16) for w in Ws_sk]

    eye16 = jnp.eye(IMU_TG, dtype=f32)
    a_imu_packed = jnp.pad(jnp.kron(eye16, A_imu),
                           ((0, 32), (0, 32))).astype(jnp.bfloat16)
    wt_imu = [w.T.astype(jnp.bfloat16) for w in Ws_imu]

    xs = jnp.transpose(skeleton, (0, 1, 4, 2, 3))
    xs = xs.reshape(B, 3, SK_GROUPS, SK_TG * V)
    xs = jnp.pad(xs, ((0, 0), (0, 0), (0, 0), (0, 28)))
    xs = xs.reshape(B, 3, SK_COLS).astype(jnp.bfloat16)

    ys = jnp.transpose(inertial, (0, 2, 1))
    ys = ys.reshape(B, IMU_GROUPS, IMU_TG * 6)
    ys = jnp.pad(ys, ((0, 0), (0, 0), (0, 32)))
    ys = ys.reshape(B, 1, IMU_COLS).astype(jnp.bfloat16)

    sk_specs = ([pl.BlockSpec((1, 3, SK_COLS), lambda i: (i, 0, 0))]
                + [_full((128, 128)) for _ in range(10)]
                + [_full(w.shape) for w in wt_sk])
    sk_feat = pl.pallas_call(
        _sk_kernel,
        grid=(B,),
        in_specs=sk_specs,
        out_specs=pl.BlockSpec((1, 1, 256), lambda i: (i, 0, 0)),
        out_shape=jax.ShapeDtypeStruct((B, 1, 256), f32),
        compiler_params=pltpu.CompilerParams(
            dimension_semantics=("parallel",)),
    )(xs, *a_sk_packed, *wt_sk)
    sk_feat = sk_feat.reshape(B, 256)

    imu_specs = ([pl.BlockSpec((1, 1, IMU_COLS), lambda i: (i, 0, 0)),
                  _full((128, 128))]
                 + [_full(w.shape) for w in wt_imu])
    imu_feat = pl.pallas_call(
        _imu_kernel,
        grid=(B,),
        in_specs=imu_specs,
        out_specs=pl.BlockSpec((1, 1, 256), lambda i: (i, 0, 0)),
        out_shape=jax.ShapeDtypeStruct((B, 1, 256), f32),
        compiler_params=pltpu.CompilerParams(
            dimension_semantics=("parallel",)),
    )(ys, a_imu_packed, *wt_imu)
    imu_feat = imu_feat.reshape(B, 256)

    out = pl.pallas_call(
        _fc_kernel,
        in_specs=[_full((B, 256)), _full((B, 256)),
                  _full((256, NUM_CLASSES)), _full((256, NUM_CLASSES)),
                  _full((1, NUM_CLASSES))],
        out_specs=_full((B, NUM_CLASSES)),
        out_shape=jax.ShapeDtypeStruct((B, NUM_CLASSES), f32),
    )(sk_feat, imu_feat, W_fc[:256], W_fc[256:], b_fc.reshape(1, NUM_CLASSES))
    return out


# speedup vs baseline: 1.9688x; 1.9688x over previous
import functools

import jax
import jax.numpy as jnp
import numpy as np
from jax.experimental import pallas as pl
from jax.experimental.pallas import tpu as pltpu

B = 16
T = 128
V = 25
M = 2
NUM_CLASSES = 27

SK_TG = 5
SK_SLOTS = 260
SK_GROUPS = SK_SLOTS // SK_TG
SK_COLS = SK_GROUPS * 128
SK_VALID = M * T * V

IMU_TG = 16
IMU_GROUPS = T // IMU_TG
IMU_COLS = IMU_GROUPS * 128
IMU_VALID = T * 6


def _sk_kernel(x_ref, *refs):
    a_refs = refs[:10]
    w_refs = refs[10:20]
    out_ref = refs[20]
    x = x_ref[0]
    for l in range(10):
        c = x.shape[0]
        a = a_refs[l][...]
        y = jnp.concatenate(
            [jax.lax.dot_general(x[:, g * 128:(g + 1) * 128], a,
                                 (((1,), (0,)), ((), ())),
                                 preferred_element_type=jnp.float32)
             .astype(jnp.bfloat16)
             for g in range(SK_GROUPS)], axis=1)
        z = jax.lax.dot_general(w_refs[l][...], y,
                                (((1,), (0,)), ((), ())),
                                preferred_element_type=jnp.float32)
        x = jnp.maximum(z.astype(jnp.bfloat16), jnp.bfloat16(0.0))
    out_ref[0, 0, :] = jnp.sum(x.astype(jnp.float32), axis=1) * (1.0 / SK_VALID)


def _imu_kernel(y_ref, a_ref, *refs):
    w_refs = refs[:5]
    out_ref = refs[5]
    y = y_ref[0]
    a = a_ref[...]
    for l in range(5):
        c = y.shape[0]
        s = jnp.concatenate(
            [jax.lax.dot_general(y[:, g * 128:(g + 1) * 128], a,
                                 (((1,), (0,)), ((), ())),
                                 preferred_element_type=jnp.float32)
             .astype(jnp.bfloat16)
             for g in range(IMU_GROUPS)], axis=1)
        z = jax.lax.dot_general(w_refs[l][...], s,
                                (((1,), (0,)), ((), ())),
                                preferred_element_type=jnp.float32)
        y = jnp.maximum(z.astype(jnp.bfloat16), jnp.bfloat16(0.0))
    out_ref[0, 0, :] = jnp.sum(y.astype(jnp.float32), axis=1) * (1.0 / IMU_VALID)


def _fc_kernel(sk_ref, imu_ref, wt_ref, wb_ref, b_ref, out_ref):
    top = jax.lax.dot_general(sk_ref[...], wt_ref[...],
                              (((1,), (0,)), ((), ())),
                              preferred_element_type=jnp.float32)
    bot = jax.lax.dot_general(imu_ref[...], wb_ref[...],
                              (((1,), (0,)), ((), ())),
                              preferred_element_type=jnp.float32)
    out_ref[...] = top + bot + b_ref[...]


def _full(shape):
    ndim = len(shape)
    return pl.BlockSpec(shape, lambda *_: (0,) * ndim)


def kernel(skeleton, inertial, A_sk, A_imu, Ws_sk, Bs_sk, Ws_imu, W_fc, b_fc):
    f32 = jnp.float32

    eye5 = jnp.eye(SK_TG, dtype=f32)
    a_sk_packed = []
    for Badp in Bs_sk:
        ahat = A_sk + Badp
        a5 = jnp.kron(eye5, ahat)
        a_sk_packed.append(jnp.pad(a5, ((0, 3), (0, 3))).astype(jnp.bfloat16))
    wt_sk = [w.T.astype(jnp.bfloat16) for w in Ws_sk]

    eye16 = jnp.eye(IMU_TG, dtype=f32)
    a_imu_packed = jnp.pad(jnp.kron(eye16, A_imu),
                           ((0, 32), (0, 32))).astype(jnp.bfloat16)
    wt_imu = [w.T.astype(jnp.bfloat16) for w in Ws_imu]

    xs = jnp.transpose(skeleton, (0, 1, 4, 2, 3))
    xs = xs.reshape(B, 3, M * T, V)
    xs = jnp.pad(xs, ((0, 0), (0, 0), (0, SK_SLOTS - M * T), (0, 0)))
    xs = xs.reshape(B, 3, SK_GROUPS, SK_TG * V)
    xs = jnp.pad(xs, ((0, 0), (0, 0), (0, 0), (0, 3)))
    xs = xs.reshape(B, 3, SK_COLS).astype(jnp.bfloat16)

    ys = jnp.transpose(inertial, (0, 2, 1))
    ys = ys.reshape(B, IMU_GROUPS, IMU_TG * 6)
    ys = jnp.pad(ys, ((0, 0), (0, 0), (0, 32)))
    ys = ys.reshape(B, 1, IMU_COLS).astype(jnp.bfloat16)

    sk_specs = ([pl.BlockSpec((1, 3, SK_COLS), lambda i: (i, 0, 0))]
                + [_full((128, 128)) for _ in range(10)]
                + [_full(w.shape) for w in wt_sk])
    sk_feat = pl.pallas_call(
        _sk_kernel,
        grid=(B,),
        in_specs=sk_specs,
        out_specs=pl.BlockSpec((1, 1, 256), lambda i: (i, 0, 0)),
        out_shape=jax.ShapeDtypeStruct((B, 1, 256), f32),
        compiler_params=pltpu.CompilerParams(
            dimension_semantics=("parallel",)),
    )(xs, *a_sk_packed, *wt_sk)
    sk_feat = sk_feat.reshape(B, 256)

    imu_specs = ([pl.BlockSpec((1, 1, IMU_COLS), lambda i: (i, 0, 0)),
                  _full((128, 128))]
                 + [_full(w.shape) for w in wt_imu])
    imu_feat = pl.pallas_call(
        _imu_kernel,
        grid=(B,),
        in_specs=imu_specs,
        out_specs=pl.BlockSpec((1, 1, 256), lambda i: (i, 0, 0)),
        out_shape=jax.ShapeDtypeStruct((B, 1, 256), f32),
        compiler_params=pltpu.CompilerParams(
            dimension_semantics=("parallel",)),
    )(ys, a_imu_packed, *wt_imu)
    imu_feat = imu_feat.reshape(B, 256)

    out = pl.pallas_call(
        _fc_kernel,
        in_specs=[_full((B, 256)), _full((B, 256)),
                  _full((256, NUM_CLASSES)), _full((256, NUM_CLASSES)),
                  _full((1, NUM_CLASSES))],
        out_specs=_full((B, NUM_CLASSES)),
        out_shape=jax.ShapeDtypeStruct((B, NUM_CLASSES), f32),
    )(sk_feat, imu_feat, W_fc[:256], W_fc[256:], b_fc.reshape(1, NUM_CLASSES))
    return out


# speedup vs baseline: 2.0347x; 1.0335x over previous
import jax
import jax.numpy as jnp
import numpy as np
from jax.experimental import pallas as pl
from jax.experimental.pallas import tpu as pltpu

B = 16
T = 128
V = 25
M = 2
NUM_CLASSES = 27

SK_TG = 5
SK_SLOTS = 260
SK_GROUPS = SK_SLOTS // SK_TG
SK_COLS = SK_GROUPS * 128
SK_VALID = M * T * V

IMU_TG = 16
IMU_GROUPS = T // IMU_TG
IMU_COLS = IMU_GROUPS * 128
IMU_VALID = T * 6

BF = jnp.bfloat16


def _spatial(x, a, groups):
    return jnp.concatenate(
        [jax.lax.dot_general(x[:, g * 128:(g + 1) * 128], a,
                             (((1,), (0,)), ((), ())),
                             preferred_element_type=jnp.float32)
         .astype(BF)
         for g in range(groups)], axis=1)


def _channel_relu(wt, x):
    z = jax.lax.dot_general(wt, x, (((1,), (0,)), ((), ())),
                            preferred_element_type=jnp.float32)
    return jnp.maximum(z.astype(BF), BF(0.0))


def _fused_kernel(xs_ref, ys_ref, *refs):
    a_sk = refs[0:10]
    w_sk = refs[10:20]
    a_imu = refs[20]
    w_imu = refs[21:26]
    wtop_ref, wbot_ref, b_ref, out_ref = refs[26:30]

    x = xs_ref[0]
    for l in range(10):
        x = _channel_relu(w_sk[l][...], _spatial(x, a_sk[l][...], SK_GROUPS))

    y = ys_ref[0]
    a = a_imu[...]
    for l in range(5):
        y = _channel_relu(w_imu[l][...], _spatial(y, a, IMU_GROUPS))

    top = jax.lax.dot_general(wtop_ref[...], x, (((1,), (0,)), ((), ())),
                              preferred_element_type=jnp.float32)
    bot = jax.lax.dot_general(wbot_ref[...], y, (((1,), (0,)), ((), ())),
                              preferred_element_type=jnp.float32)
    out = (jnp.sum(top, axis=1) * (1.0 / SK_VALID)
           + jnp.sum(bot, axis=1) * (1.0 / IMU_VALID) + b_ref[0, :])
    out_ref[0, 0, :] = out


def _full(shape):
    ndim = len(shape)
    return pl.BlockSpec(shape, lambda *_: (0,) * ndim)


def kernel(skeleton, inertial, A_sk, A_imu, Ws_sk, Bs_sk, Ws_imu, W_fc, b_fc):
    f32 = jnp.float32

    eye5 = jnp.eye(SK_TG, dtype=f32)
    a_sk_packed = []
    for Badp in Bs_sk:
        ahat = A_sk + Badp
        a5 = jnp.kron(eye5, ahat)
        a_sk_packed.append(jnp.pad(a5, ((0, 3), (0, 3))).astype(BF))
    wt_sk = [w.T.astype(BF) for w in Ws_sk]

    eye16 = jnp.eye(IMU_TG, dtype=f32)
    a_imu_packed = jnp.pad(jnp.kron(eye16, A_imu),
                           ((0, 32), (0, 32))).astype(BF)
    wt_imu = [w.T.astype(BF) for w in Ws_imu]

    wtop = W_fc[:256].T.astype(BF)
    wbot = W_fc[256:].T.astype(BF)
    b2 = b_fc.reshape(1, NUM_CLASSES)

    xs = jnp.transpose(skeleton, (0, 1, 4, 2, 3))
    xs = xs.reshape(B, 3, M * T, V)
    xs = jnp.pad(xs, ((0, 0), (0, 0), (0, SK_SLOTS - M * T), (0, 0)))
    xs = xs.reshape(B, 3, SK_GROUPS, SK_TG * V)
    xs = jnp.pad(xs, ((0, 0), (0, 0), (0, 0), (0, 3)))
    xs = xs.reshape(B, 3, SK_COLS).astype(BF)

    ys = jnp.transpose(inertial, (0, 2, 1))
    ys = ys.reshape(B, IMU_GROUPS, IMU_TG * 6)
    ys = jnp.pad(ys, ((0, 0), (0, 0), (0, 32)))
    ys = ys.reshape(B, 1, IMU_COLS).astype(BF)

    in_specs = ([pl.BlockSpec((1, 3, SK_COLS), lambda i: (i, 0, 0)),
                 pl.BlockSpec((1, 1, IMU_COLS), lambda i: (i, 0, 0))]
                + [_full((128, 128)) for _ in range(10)]
                + [_full(w.shape) for w in wt_sk]
                + [_full((128, 128))]
                + [_full(w.shape) for w in wt_imu]
                + [_full((NUM_CLASSES, 256)), _full((NUM_CLASSES, 256)),
                   _full((1, NUM_CLASSES))])
    out = pl.pallas_call(
        _fused_kernel,
        grid=(B,),
        in_specs=in_specs,
        out_specs=pl.BlockSpec((1, 1, NUM_CLASSES), lambda i: (i, 0, 0)),
        out_shape=jax.ShapeDtypeStruct((B, 1, NUM_CLASSES), f32),
        compiler_params=pltpu.CompilerParams(
            dimension_semantics=("parallel",)),
    )(xs, ys, *a_sk_packed, *wt_sk, a_imu_packed, *wt_imu, wtop, wbot, b2)
    return out.reshape(B, NUM_CLASSES)


# speedup vs baseline: 2.1454x; 1.0544x over previous
import jax
import jax.numpy as jnp
import numpy as np
from jax.experimental import pallas as pl
from jax.experimental.pallas import tpu as pltpu

B = 16
T = 128
V = 25
M = 2
NUM_CLASSES = 27

SK_TG = 5
SK_SLOTS = 260
SK_GROUPS = SK_SLOTS // SK_TG
SK_COLS = SK_GROUPS * 128
SK_VALID = M * T * V

IMU_TG = 16
IMU_GROUPS = T // IMU_TG
IMU_COLS = IMU_GROUPS * 128
IMU_VALID = T * 6

BF = jnp.bfloat16

SPP = 2
PROGS = B // SPP


def _spatial(x, a, groups):
    return jnp.concatenate(
        [jax.lax.dot_general(x[:, g * 128:(g + 1) * 128], a,
                             (((1,), (0,)), ((), ())),
                             preferred_element_type=jnp.float32)
         .astype(BF)
         for g in range(groups)], axis=1)


def _channel_relu(wt, x):
    z = jax.lax.dot_general(wt, x, (((1,), (0,)), ((), ())),
                            preferred_element_type=jnp.float32)
    return jnp.maximum(z.astype(BF), BF(0.0))


def _fused_kernel(xs_ref, ys_ref, *refs):
    a_sk = refs[0:10]
    w_sk = refs[10:20]
    a_imu = refs[20]
    w_imu = refs[21:26]
    wtop_ref, wbot_ref, b_ref, out_ref = refs[26:30]

    x = xs_ref[0]
    for l in range(10):
        x = _channel_relu(w_sk[l][...], _spatial(x, a_sk[l][...],
                                                 SPP * SK_GROUPS))

    y = ys_ref[0]
    a = a_imu[...]
    for l in range(5):
        y = _channel_relu(w_imu[l][...], _spatial(y, a, SPP * IMU_GROUPS))

    top = jax.lax.dot_general(wtop_ref[...], x, (((1,), (0,)), ((), ())),
                              preferred_element_type=jnp.float32)
    bot = jax.lax.dot_general(wbot_ref[...], y, (((1,), (0,)), ((), ())),
                              preferred_element_type=jnp.float32)
    for s in range(SPP):
        out = (jnp.sum(top[:, s * SK_COLS:(s + 1) * SK_COLS], axis=1)
               * (1.0 / SK_VALID)
               + jnp.sum(bot[:, s * IMU_COLS:(s + 1) * IMU_COLS], axis=1)
               * (1.0 / IMU_VALID) + b_ref[0, :])
        out_ref[0, s, :] = out


def _full(shape):
    ndim = len(shape)
    return pl.BlockSpec(shape, lambda *_: (0,) * ndim)


def kernel(skeleton, inertial, A_sk, A_imu, Ws_sk, Bs_sk, Ws_imu, W_fc, b_fc):
    f32 = jnp.float32

    eye5 = jnp.eye(SK_TG, dtype=f32)
    a_sk_packed = []
    for Badp in Bs_sk:
        ahat = A_sk + Badp
        a5 = jnp.kron(eye5, ahat)
        a_sk_packed.append(jnp.pad(a5, ((0, 3), (0, 3))).astype(BF))
    wt_sk = [w.T.astype(BF) for w in Ws_sk]

    eye16 = jnp.eye(IMU_TG, dtype=f32)
    a_imu_packed = jnp.pad(jnp.kron(eye16, A_imu),
                           ((0, 32), (0, 32))).astype(BF)
    wt_imu = [w.T.astype(BF) for w in Ws_imu]

    wtop = W_fc[:256].T.astype(BF)
    wbot = W_fc[256:].T.astype(BF)
    b2 = b_fc.reshape(1, NUM_CLASSES)

    xs = jnp.transpose(skeleton, (0, 1, 4, 2, 3))
    xs = xs.reshape(B, 3, M * T, V)
    xs = jnp.pad(xs, ((0, 0), (0, 0), (0, SK_SLOTS - M * T), (0, 0)))
    xs = xs.reshape(B, 3, SK_GROUPS, SK_TG * V)
    xs = jnp.pad(xs, ((0, 0), (0, 0), (0, 0), (0, 3)))
    xs = xs.reshape(PROGS, SPP, 3, SK_COLS).transpose(0, 2, 1, 3)
    xs = xs.reshape(PROGS, 3, SPP * SK_COLS).astype(BF)

    ys = jnp.transpose(inertial, (0, 2, 1))
    ys = ys.reshape(B, IMU_GROUPS, IMU_TG * 6)
    ys = jnp.pad(ys, ((0, 0), (0, 0), (0, 32)))
    ys = ys.reshape(PROGS, 1, SPP * IMU_COLS).astype(BF)

    in_specs = ([pl.BlockSpec((1, 3, SPP * SK_COLS), lambda i: (i, 0, 0)),
                 pl.BlockSpec((1, 1, SPP * IMU_COLS), lambda i: (i, 0, 0))]
                + [_full((128, 128)) for _ in range(10)]
                + [_full(w.shape) for w in wt_sk]
                + [_full((128, 128))]
                + [_full(w.shape) for w in wt_imu]
                + [_full((NUM_CLASSES, 256)), _full((NUM_CLASSES, 256)),
                   _full((1, NUM_CLASSES))])
    out = pl.pallas_call(
        _fused_kernel,
        grid=(PROGS,),
        in_specs=in_specs,
        out_specs=pl.BlockSpec((1, SPP, NUM_CLASSES), lambda i: (i, 0, 0)),
        out_shape=jax.ShapeDtypeStruct((PROGS, SPP, NUM_CLASSES), f32),
        compiler_params=pltpu.CompilerParams(
            dimension_semantics=("parallel",)),
    )(xs, ys, *a_sk_packed, *wt_sk, a_imu_packed, *wt_imu, wtop, wbot, b2)
    return out.reshape(B, NUM_CLASSES)
